# Initial kernel scaffold; baseline (speedup 1.0000x reference)
import jax
import jax.numpy as jnp
from jax.experimental import pallas as pl


def kernel(x, gate_w, gate_b, W1, b1, W2, b2):
    raise NotImplementedError("write your pallas kernel here")


# speedup vs baseline: 3.0126x; 3.0126x over previous
import functools

import jax
import jax.numpy as jnp
from jax import lax
from jax.experimental import pallas as pl
from jax.experimental.pallas import tpu as pltpu

H = 768
E = 8
FF = 4 * H
T = 2048
NEG = -1e30


def _gating_body(x_ref, gw_ref, gb_ref, wmat_ref, laux_ref, counts_ref):
    x = x_ref[...]
    gw = gw_ref[...]
    gb = gb_ref[...]
    lt = lax.dot_general(gw, x, (((1,), (1,)), ((), ())),
                         preferred_element_type=jnp.float32) + gb
    rows = lax.broadcasted_iota(jnp.int32, (E, T), 0)
    m1 = jnp.max(lt, axis=0, keepdims=True)
    e1 = jnp.min(jnp.where(lt == m1, rows, E), axis=0, keepdims=True)
    mask1 = (rows == e1)
    lt2 = jnp.where(mask1, NEG, lt)
    m2 = jnp.max(lt2, axis=0, keepdims=True)
    e2 = jnp.min(jnp.where(lt2 == m2, rows, E), axis=0, keepdims=True)
    mask2 = (rows == e2)
    w1 = 1.0 / (1.0 + jnp.exp(m2 - m1))
    w2 = 1.0 - w1
    wmat = jnp.where(mask1, w1, 0.0) + jnp.where(mask2, w2, 0.0)
    wmat_ref[...] = wmat
    cnt = jnp.sum(mask1.astype(jnp.float32) + mask2.astype(jnp.float32),
                  axis=1, keepdims=True)
    counts_ref[...] = cnt
    p = jnp.exp(lt - jnp.max(lt, axis=0, keepdims=True))
    p = p / jnp.sum(p, axis=0, keepdims=True)
    pm = jnp.mean(p, axis=1, keepdims=True)
    laux_ref[0, 0] = jnp.sum(pm * pm) * E


def _gating(x2d, gate_w, gate_b):
    return pl.pallas_call(
        _gating_body,
        out_shape=(
            jax.ShapeDtypeStruct((E, T), jnp.float32),
            jax.ShapeDtypeStruct((1, 1), jnp.float32),
            jax.ShapeDtypeStruct((E, 1), jnp.float32),
        ),
        out_specs=(
            pl.BlockSpec(memory_space=pltpu.VMEM),
            pl.BlockSpec(memory_space=pltpu.SMEM),
            pl.BlockSpec(memory_space=pltpu.VMEM),
        ),
    )(x2d, gate_w, gate_b.reshape(E, 1))


NJ = 4
FBLK = FF // NJ


def _ffn_body(x_ref, w1_ref, b1_ref, w2_ref, b2_ref, wt_ref, out_ref):
    e = pl.program_id(0)
    j = pl.program_id(1)
    x = x_ref[...]
    h = lax.dot_general(x, w1_ref[0], (((1,), (1,)), ((), ())),
                        preferred_element_type=jnp.float32)
    h = h + b1_ref[0]
    h = 0.5 * h * (1.0 + lax.erf(h * 0.7071067811865476))
    part = lax.dot_general(h, w2_ref[0], (((1,), (1,)), ((), ())),
                           preferred_element_type=jnp.float32)
    onehot = (lax.broadcasted_iota(jnp.int32, (E, 1), 0) == e).astype(jnp.float32)
    wcol = lax.dot_general(wt_ref[...], onehot, (((0,), (0,)), ((), ())),
                           preferred_element_type=jnp.float32)
    contrib = part * wcol
    contrib = jnp.where(j == 0, contrib + b2_ref[0] * wcol, contrib)
    first = jnp.logical_and(e == 0, j == 0)
    prev = jnp.where(first, 0.0, out_ref[...])
    out_ref[...] = prev + contrib


def _ffn(x2d, W1, b1, W2, b2, wmat):
    return pl.pallas_call(
        _ffn_body,
        grid=(E, NJ),
        in_specs=[
            pl.BlockSpec((T, H), lambda e, j: (0, 0)),
            pl.BlockSpec((1, FBLK, H), lambda e, j: (e, j, 0)),
            pl.BlockSpec((1, 1, FBLK), lambda e, j: (e, 0, j)),
            pl.BlockSpec((1, H, FBLK), lambda e, j: (e, 0, j)),
            pl.BlockSpec((1, 1, H), lambda e, j: (e, 0, 0)),
            pl.BlockSpec((E, T), lambda e, j: (0, 0)),
        ],
        out_specs=pl.BlockSpec((T, H), lambda e, j: (0, 0)),
        out_shape=jax.ShapeDtypeStruct((T, H), jnp.float32),
    )(x2d, W1, b1.reshape(E, 1, FF), W2, b2.reshape(E, 1, H), wmat)


def kernel(x, gate_w, gate_b, W1, b1, W2, b2):
    bsz, seq, hidden = x.shape
    x2d = x.reshape(T, H)
    wmat, laux, counts = _gating(x2d, gate_w, gate_b)
    out2d = _ffn(x2d, W1, b1, W2, b2, wmat)
    return out2d.reshape(bsz, seq, hidden), laux[0, 0], counts.reshape(E)
